# Initial kernel scaffold; baseline (speedup 1.0000x reference)
import jax
import jax.numpy as jnp
from jax.experimental import pallas as pl


def kernel(embs, edge_index):
    raise NotImplementedError("write your pallas kernel here")


# speedup vs baseline: 6.2255x; 6.2255x over previous
import jax
import jax.numpy as jnp
from jax import lax
from jax.experimental import pallas as pl
from jax.experimental.pallas import tpu as pltpu
from jax.experimental.pallas import tpu_sc as plsc

N = 10000
E = 320000
D = 128
H = D // 2
NC = 2
NS = 16
NDIS = 10240
SL = NDIS // NS
CE = 128
NCHUNK = E // CE
KMAX = -(-NCHUNK // NS)
NRC = N // 16
KROW = -(-NRC // NS)

_mesh = plsc.VectorSubcoreMesh(core_axis_name="c", subcore_axis_name="s")
_params = pltpu.CompilerParams(use_tc_tiling_on_sc=False)


def _rsqrt16(d):
  i = lax.bitcast_convert_type(d, jnp.int32)
  i = jnp.int32(0x5F3759DF) - lax.shift_right_logical(i, 1)
  y = lax.bitcast_convert_type(i, jnp.float32)
  for _ in range(3):
    t = (d * 0.5) * y
    y = y * (1.5 - t * y)
  return y


def _bcast_row(dvec, n):
  return jnp.full((16,), dvec[n], jnp.float32)


def _setup_body(embs, col, t0, dis_out, deg_sh, dis_sh, dbuf, ones_v,
                cidx_v, ev, tv, disc):
  c = lax.axis_index("c")
  s = lax.axis_index("s")

  for j in range(SL // 16):
    dbuf[pl.ds(j * 16, 16)] = jnp.zeros((16,), jnp.float32)
  for j in range(CE // 16):
    ones_v[pl.ds(j * 16, 16)] = jnp.ones((16,), jnp.float32)
  pltpu.sync_copy(dbuf, deg_sh.at[pl.ds(s * SL, SL)])
  plsc.subcore_barrier()

  @pl.loop(0, KMAX)
  def _deg(k):
    cid = s + NS * k

    @pl.when(cid < NCHUNK)
    def _():
      pltpu.sync_copy(col.at[pl.ds(cid * CE, CE)], cidx_v)
      pltpu.sync_copy(ones_v, deg_sh.at[cidx_v], add=True)

  plsc.subcore_barrier()

  pltpu.sync_copy(deg_sh.at[pl.ds(s * SL, SL)], dbuf)
  for j in range(SL // 16):
    d = dbuf[pl.ds(j * 16, 16)]
    y = jnp.where(d > 0, _rsqrt16(d), 0.0)
    dbuf[pl.ds(j * 16, 16)] = y
  pltpu.sync_copy(dbuf, dis_sh.at[pl.ds(s * SL, SL)])

  @pl.when(c == 0)
  def _():
    pltpu.sync_copy(dbuf, dis_out.at[pl.ds(s * SL, SL)])

  plsc.subcore_barrier()

  @pl.loop(0, KROW)
  def _rows(k):
    cid = s + NS * k

    @pl.when(cid < NRC)
    def _():
      r0 = cid * 16
      pltpu.sync_copy(embs.at[c, pl.ds(r0, 16)], ev)
      pltpu.sync_copy(dis_sh.at[pl.ds(r0, 16)], disc)
      dvec = disc[...]
      for n in range(16):
        d16 = _bcast_row(dvec, n)
        for j in range(H // 16):
          tv[n, pl.ds(j * 16, 16)] = ev[n, pl.ds(j * 16, 16)] * d16
      pltpu.sync_copy(tv, t0.at[c, pl.ds(r0, 16)])


_setup = pl.kernel(
    _setup_body,
    out_type=[
        jax.ShapeDtypeStruct((NC, N, H), jnp.float32),
        jax.ShapeDtypeStruct((NDIS,), jnp.float32),
    ],
    mesh=_mesh,
    compiler_params=_params,
    scratch_types=[
        pltpu.VMEM_SHARED((NDIS,), jnp.float32),
        pltpu.VMEM_SHARED((NDIS,), jnp.float32),
        pltpu.VMEM((SL,), jnp.float32),
        pltpu.VMEM((CE,), jnp.float32),
        pltpu.VMEM((CE,), jnp.int32),
        pltpu.VMEM((16, H), jnp.float32),
        pltpu.VMEM((16, H), jnp.float32),
        pltpu.VMEM((16,), jnp.float32),
    ],
)


def _layer_body(final, t, row, col, dis_hbm, accin, *rest):
  if final:
    (out_h, acc_sh, rowv, colv, gbuf, av, pv, tnv, ov, disc, sem) = rest
    t_next = None
  else:
    (t_next, out_h, acc_sh, rowv, colv, gbuf, av, pv, tnv, ov, disc,
     sem) = rest
  c = lax.axis_index("c")
  s = lax.axis_index("s")

  for r in range(CE):
    for j in range(H // 16):
      gbuf[r, pl.ds(j * 16, 16)] = jnp.zeros((16,), jnp.float32)
  for q in range(N // NS // CE):
    pltpu.sync_copy(gbuf, acc_sh.at[pl.ds(s * (N // NS) + q * CE, CE)])
  rem = N // NS - (N // NS // CE) * CE
  pltpu.sync_copy(gbuf.at[pl.ds(0, rem)],
                  acc_sh.at[pl.ds(s * (N // NS) + (N // NS // CE) * CE, rem)])
  plsc.subcore_barrier()

  tc = t.at[c]

  @pl.loop(0, KMAX)
  def _edges(k):
    cid = s + NS * k

    @pl.when(cid < NCHUNK)
    def _():
      base = cid * CE
      pltpu.sync_copy(row.at[pl.ds(base, CE)], rowv)
      pltpu.sync_copy(col.at[pl.ds(base, CE)], colv)
      pltpu.async_copy(tc.at[rowv], gbuf, sem).wait()
      pltpu.sync_copy(gbuf, acc_sh.at[colv], add=True)

  plsc.subcore_barrier()

  @pl.loop(0, KROW)
  def _rows(k):
    cid = s + NS * k

    @pl.when(cid < NRC)
    def _():
      r0 = cid * 16
      pltpu.sync_copy(acc_sh.at[pl.ds(r0, 16)], av)
      pltpu.sync_copy(dis_hbm.at[pl.ds(r0, 16)], disc)
      pltpu.sync_copy(accin.at[c, pl.ds(r0, 16)], pv)
      dvec = disc[...]
      for n in range(16):
        d16 = _bcast_row(dvec, n)
        for j in range(H // 16):
          sl = pl.ds(j * 16, 16)
          x = av[n, sl] * d16
          if final:
            ov[n, sl] = (pv[n, sl] + x) * 0.25
          else:
            ov[n, sl] = pv[n, sl] + x
            tnv[n, sl] = x * d16
      pltpu.sync_copy(ov, out_h.at[c, pl.ds(r0, 16)])
      if not final:
        pltpu.sync_copy(tnv, t_next.at[c, pl.ds(r0, 16)])


def _make_layer(final):
  outs = []
  if not final:
    outs.append(jax.ShapeDtypeStruct((NC, N, H), jnp.float32))
  outs.append(jax.ShapeDtypeStruct((NC, N, H), jnp.float32))
  return pl.kernel(
      lambda *a: _layer_body(final, *a),
      out_type=outs,
      mesh=_mesh,
      compiler_params=_params,
      scratch_types=[
          pltpu.VMEM_SHARED((N, H), jnp.float32),
          pltpu.VMEM((CE,), jnp.int32),
          pltpu.VMEM((CE,), jnp.int32),
          pltpu.VMEM((CE, H), jnp.float32),
          pltpu.VMEM((16, H), jnp.float32),
          pltpu.VMEM((16, H), jnp.float32),
          pltpu.VMEM((16, H), jnp.float32),
          pltpu.VMEM((16, H), jnp.float32),
          pltpu.VMEM((16,), jnp.float32),
          pltpu.SemaphoreType.DMA,
      ],
  )


_layer_mid = _make_layer(final=False)
_layer_final = _make_layer(final=True)


def kernel(embs, edge_index):
  row = edge_index[0]
  col = edge_index[1]
  embs2 = jnp.stack([embs[:, :H], embs[:, H:]])
  t0, dis = _setup(embs2, col)
  t1, acc1 = _layer_mid(t0, row, col, dis, embs2)
  t2, acc2 = _layer_mid(t1, row, col, dis, acc1)
  (out_h,) = _layer_final(t2, row, col, dis, acc2)
  return jnp.concatenate([out_h[0], out_h[1]], axis=1)


# speedup vs baseline: 13.8585x; 2.2261x over previous
import jax
import jax.numpy as jnp
from jax import lax
from jax.experimental import pallas as pl
from jax.experimental.pallas import tpu as pltpu
from jax.experimental.pallas import tpu_sc as plsc

N = 10000
E = 320000
D = 128
H = D // 2
NC = 2
NS = 16
NDIS = 10240
NACC = 10240
SL = NDIS // NS
CE = 128
KMAX = 158
NCHUNKP = KMAX * NS
EP = NCHUNKP * CE
NRC = N // 16
KROW = -(-NRC // NS)

_mesh = plsc.VectorSubcoreMesh(core_axis_name="c", subcore_axis_name="s")
_params = pltpu.CompilerParams(use_tc_tiling_on_sc=False)


def _rsqrt16(d):
  i = lax.bitcast_convert_type(d, jnp.int32)
  i = jnp.int32(0x5F3759DF) - lax.shift_right_logical(i, 1)
  y = lax.bitcast_convert_type(i, jnp.float32)
  for _ in range(3):
    t = (d * 0.5) * y
    y = y * (1.5 - t * y)
  return y


def _bcast_row(dvec, n):
  return jnp.full((16,), dvec[n], jnp.float32)


def _zero_vmem_2d(ref, rows, width):
  for r in range(rows):
    for j in range(width // 16):
      ref[r, pl.ds(j * 16, 16)] = jnp.zeros((16,), jnp.float32)


def _setup_body(embs, col2, t0, dis_out, deg_sh, dis_sh, dbuf, ones_v,
                colv2, ev, tv, disc):
  c = lax.axis_index("c")
  s = lax.axis_index("s")

  for j in range(SL // 16):
    dbuf[pl.ds(j * 16, 16)] = jnp.zeros((16,), jnp.float32)
  for j in range(CE // 16):
    ones_v[pl.ds(j * 16, 16)] = jnp.ones((16,), jnp.float32)
  pltpu.sync_copy(dbuf, deg_sh.at[pl.ds(s * SL, SL)])
  pltpu.sync_copy(col2.at[pl.ds(s * KMAX, KMAX)], colv2)
  plsc.subcore_barrier()

  @pl.loop(0, KMAX)
  def _deg(k):
    pltpu.sync_copy(ones_v, deg_sh.at[colv2.at[k]], add=True)

  plsc.subcore_barrier()

  pltpu.sync_copy(deg_sh.at[pl.ds(s * SL, SL)], dbuf)
  for j in range(SL // 16):
    d = dbuf[pl.ds(j * 16, 16)]
    y = jnp.where(d > 0, _rsqrt16(d), 0.0)
    dbuf[pl.ds(j * 16, 16)] = y
  pltpu.sync_copy(dbuf, dis_sh.at[pl.ds(s * SL, SL)])

  @pl.when(c == 0)
  def _():
    pltpu.sync_copy(dbuf, dis_out.at[pl.ds(s * SL, SL)])

  plsc.subcore_barrier()

  @pl.loop(0, KROW)
  def _rows(k):
    cid = s + NS * k

    @pl.when(cid < NRC)
    def _():
      r0 = cid * 16
      pltpu.sync_copy(embs.at[c, pl.ds(r0, 16)], ev)
      pltpu.sync_copy(dis_sh.at[pl.ds(r0, 16)], disc)
      dvec = disc[...]
      for n in range(16):
        d16 = _bcast_row(dvec, n)
        for j in range(H // 16):
          tv[n, pl.ds(j * 16, 16)] = ev[n, pl.ds(j * 16, 16)] * d16
      pltpu.sync_copy(tv, t0.at[c, pl.ds(r0, 16)])


_setup = pl.kernel(
    _setup_body,
    out_type=[
        jax.ShapeDtypeStruct((NC, N, H), jnp.float32),
        jax.ShapeDtypeStruct((NDIS,), jnp.float32),
    ],
    mesh=_mesh,
    compiler_params=_params,
    scratch_types=[
        pltpu.VMEM_SHARED((NDIS,), jnp.float32),
        pltpu.VMEM_SHARED((NDIS,), jnp.float32),
        pltpu.VMEM((SL,), jnp.float32),
        pltpu.VMEM((CE,), jnp.float32),
        pltpu.VMEM((KMAX, CE), jnp.int32),
        pltpu.VMEM((16, H), jnp.float32),
        pltpu.VMEM((16, H), jnp.float32),
        pltpu.VMEM((16,), jnp.float32),
    ],
)


def _layer_body(final, t, row2, col2, dis_hbm, accin, *rest):
  if final:
    (out_h, acc_sh, rowv2, colv2, gbuf0, gbuf1, av, pv, tnv, ov, disc,
     sem0, sem1) = rest
    t_next = None
  else:
    (t_next, out_h, acc_sh, rowv2, colv2, gbuf0, gbuf1, av, pv, tnv, ov,
     disc, sem0, sem1) = rest
  c = lax.axis_index("c")
  s = lax.axis_index("s")
  tc = t.at[c]

  pltpu.sync_copy(row2.at[pl.ds(s * KMAX, KMAX)], rowv2)
  pltpu.sync_copy(col2.at[pl.ds(s * KMAX, KMAX)], colv2)
  _zero_vmem_2d(gbuf0, CE, H)
  for q in range(NACC // NS // CE):
    pltpu.sync_copy(gbuf0, acc_sh.at[pl.ds(s * (NACC // NS) + q * CE, CE)])
  plsc.subcore_barrier()

  pltpu.async_copy(tc.at[rowv2.at[0]], gbuf0, sem0)

  @pl.loop(0, KMAX, step=2)
  def _edges(g):
    pltpu.async_copy(tc.at[rowv2.at[g + 1]], gbuf1, sem1)
    pltpu.make_async_copy(tc.at[rowv2.at[g]], gbuf0, sem0).wait()
    pltpu.sync_copy(gbuf0, acc_sh.at[colv2.at[g]], add=True)

    @pl.when(g + 2 < KMAX)
    def _():
      pltpu.async_copy(tc.at[rowv2.at[g + 2]], gbuf0, sem0)

    pltpu.make_async_copy(tc.at[rowv2.at[g + 1]], gbuf1, sem1).wait()
    pltpu.sync_copy(gbuf1, acc_sh.at[colv2.at[g + 1]], add=True)

  plsc.subcore_barrier()

  @pl.loop(0, KROW)
  def _rows(k):
    cid = s + NS * k

    @pl.when(cid < NRC)
    def _():
      r0 = cid * 16
      pltpu.sync_copy(acc_sh.at[pl.ds(r0, 16)], av)
      pltpu.sync_copy(dis_hbm.at[pl.ds(r0, 16)], disc)
      pltpu.sync_copy(accin.at[c, pl.ds(r0, 16)], pv)
      dvec = disc[...]
      for n in range(16):
        d16 = _bcast_row(dvec, n)
        for j in range(H // 16):
          sl = pl.ds(j * 16, 16)
          x = av[n, sl] * d16
          if final:
            ov[n, sl] = (pv[n, sl] + x) * 0.25
          else:
            ov[n, sl] = pv[n, sl] + x
            tnv[n, sl] = x * d16
      pltpu.sync_copy(ov, out_h.at[c, pl.ds(r0, 16)])
      if not final:
        pltpu.sync_copy(tnv, t_next.at[c, pl.ds(r0, 16)])


def _make_layer(final):
  outs = []
  if not final:
    outs.append(jax.ShapeDtypeStruct((NC, N, H), jnp.float32))
  outs.append(jax.ShapeDtypeStruct((NC, N, H), jnp.float32))
  return pl.kernel(
      lambda *a: _layer_body(final, *a),
      out_type=outs,
      mesh=_mesh,
      compiler_params=_params,
      scratch_types=[
          pltpu.VMEM_SHARED((NACC, H), jnp.float32),
          pltpu.VMEM((KMAX, CE), jnp.int32),
          pltpu.VMEM((KMAX, CE), jnp.int32),
          pltpu.VMEM((CE, H), jnp.float32),
          pltpu.VMEM((CE, H), jnp.float32),
          pltpu.VMEM((16, H), jnp.float32),
          pltpu.VMEM((16, H), jnp.float32),
          pltpu.VMEM((16, H), jnp.float32),
          pltpu.VMEM((16, H), jnp.float32),
          pltpu.VMEM((16,), jnp.float32),
          pltpu.SemaphoreType.DMA,
          pltpu.SemaphoreType.DMA,
      ],
  )


_layer_mid = _make_layer(final=False)
_layer_final = _make_layer(final=True)


def kernel(embs, edge_index):
  row = edge_index[0]
  col = edge_index[1]
  npad = EP - E
  ar = jnp.arange(npad, dtype=jnp.int32)
  padrow = (ar * 97) % N
  padcol = N + (ar % (NACC - N))
  row2 = jnp.concatenate([row, padrow]).reshape(NCHUNKP, CE)
  col2 = jnp.concatenate([col, padcol]).reshape(NCHUNKP, CE)
  embs2 = jnp.stack([embs[:, :H], embs[:, H:]])
  t0, dis = _setup(embs2, col2)
  t1, acc1 = _layer_mid(t0, row2, col2, dis, embs2)
  t2, acc2 = _layer_mid(t1, row2, col2, dis, acc1)
  (out_h,) = _layer_final(t2, row2, col2, dis, acc2)
  return jnp.concatenate([out_h[0], out_h[1]], axis=1)


# speedup vs baseline: 18.2887x; 1.3197x over previous
import jax
import jax.numpy as jnp
from jax import lax
from jax.experimental import pallas as pl
from jax.experimental.pallas import tpu as pltpu
from jax.experimental.pallas import tpu_sc as plsc

N = 10000
E = 320000
D = 128
H = D // 2
NC = 2
NS = 16
NACC = 10112
NRCH = NACC // 128
NRC64 = NACC // 64
NDIS = 10240
SL = NDIS // NS
CE = 128
KMAX = 160
NCHUNKP = KMAX * NS
EP = NCHUNKP * CE

_mesh = plsc.VectorSubcoreMesh(core_axis_name="c", subcore_axis_name="s")
_params = pltpu.CompilerParams(use_tc_tiling_on_sc=False)


def _rsqrt16(d):
  i = lax.bitcast_convert_type(d, jnp.int32)
  i = jnp.int32(0x5F3759DF) - lax.shift_right_logical(i, 1)
  y = lax.bitcast_convert_type(i, jnp.float32)
  for _ in range(3):
    t = (d * 0.5) * y
    y = y * (1.5 - t * y)
  return y


def _bcast_row(dvec, n):
  return jnp.full((16,), dvec[n], jnp.float32)


def _fused_body(embs2, row2, col2, out_h, tbuf, accb,
                deg_sh, acc_sh, rowv2, colv2, b0, b1, b2, b3, av, pv,
                dbuf, disc, ones_v, g0, g1, g2, g3, ssem, dsem):
  c = lax.axis_index("c")
  s = lax.axis_index("s")
  tc = tbuf.at[c]
  bufs = (b0, b1, b2, b3)
  gsems = (g0, g1, g2, g3)

  def fire_gather(k, b):
    pltpu.async_copy(tc.at[rowv2.at[k]], bufs[b], gsems[b])

  def wait_gather(k, b):
    pltpu.make_async_copy(tc.at[rowv2.at[k]], bufs[b], gsems[b]).wait()

  def fire_scatter(k, b):
    pltpu.async_copy(bufs[b], acc_sh.at[colv2.at[k]], ssem, add=True)

  def drain_scatter(k, b):
    pltpu.make_async_copy(bufs[b], acc_sh.at[colv2.at[k]], ssem).wait()

  def zero_acc():
    for r in range(CE):
      for j in range(H // 16):
        b0[r, pl.ds(j * 16, 16)] = jnp.zeros((16,), jnp.float32)
    for q in range(5):
      cq = s + NS * q

      @pl.when(cq < NRCH)
      def _(cq=cq):
        pltpu.sync_copy(b0, acc_sh.at[pl.ds(cq * 128, CE)])

  pltpu.sync_copy(row2.at[pl.ds(s * KMAX, KMAX)], rowv2)
  pltpu.sync_copy(col2.at[pl.ds(s * KMAX, KMAX)], colv2)
  for j in range(SL // 16):
    dbuf[pl.ds(j * 16, 16)] = jnp.zeros((16,), jnp.float32)
  for j in range(CE // 16):
    ones_v[pl.ds(j * 16, 16)] = jnp.ones((16,), jnp.float32)
  pltpu.sync_copy(dbuf, deg_sh.at[pl.ds(s * SL, SL)])
  zero_acc()
  plsc.subcore_barrier()

  @pl.loop(0, KMAX, step=8)
  def _deg(k):
    for b in range(8):
      pltpu.async_copy(ones_v, deg_sh.at[colv2.at[k + b]], dsem, add=True)
    for b in range(8):
      pltpu.make_async_copy(ones_v, deg_sh.at[colv2.at[k]], dsem).wait()

  plsc.subcore_barrier()

  pltpu.sync_copy(deg_sh.at[pl.ds(s * SL, SL)], dbuf)
  for j in range(SL // 16):
    d = dbuf[pl.ds(j * 16, 16)]
    y = jnp.where(d > 0, _rsqrt16(d), 0.0)
    dbuf[pl.ds(j * 16, 16)] = y
  pltpu.sync_copy(dbuf, deg_sh.at[pl.ds(s * SL, SL)])
  plsc.subcore_barrier()

  @pl.loop(0, 10)
  def _t0(q):
    cq = s + NS * q

    @pl.when(cq < NRC64)
    def _():
      r0 = cq * 64
      pltpu.sync_copy(embs2.at[c, pl.ds(r0, 64)], av)
      pltpu.sync_copy(deg_sh.at[pl.ds(r0, 64)], disc)

      @pl.loop(0, 4)
      def _grp(gi):
        o = pl.multiple_of(gi * 16, 16)
        dvec = disc[pl.ds(o, 16)]
        for l in range(16):
          n = o + l
          d16 = _bcast_row(dvec, l)
          for j in range(H // 16):
            sl = pl.ds(j * 16, 16)
            av[n, sl] = av[n, sl] * d16

      pltpu.sync_copy(av, tc.at[pl.ds(r0, 64)])

  plsc.subcore_barrier()

  for li in range(3):
    final = li == 2

    for b in range(2):
      fire_gather(b, b)

    @pl.loop(0, KMAX, step=4)
    def _edges(g):
      wait_gather(g, 0)
      wait_gather(g + 1, 1)
      fire_scatter(g, 0)
      fire_scatter(g + 1, 1)
      fire_gather(g + 2, 2)
      fire_gather(g + 3, 3)
      drain_scatter(g, 0)
      drain_scatter(g + 1, 1)
      wait_gather(g + 2, 2)
      wait_gather(g + 3, 3)
      fire_scatter(g + 2, 2)
      fire_scatter(g + 3, 3)

      @pl.when(g + 4 < KMAX)
      def _():
        fire_gather(g + 4, 0)
        fire_gather(g + 5, 1)

      drain_scatter(g + 2, 2)
      drain_scatter(g + 3, 3)

    plsc.subcore_barrier()

    @pl.loop(0, 10)
    def _rows(q):
      cq = s + NS * q

      @pl.when(cq < NRC64)
      def _():
        r0 = cq * 64
        pltpu.sync_copy(acc_sh.at[pl.ds(r0, 64)], av)
        pltpu.sync_copy(deg_sh.at[pl.ds(r0, 64)], disc)
        if li == 0:
          pltpu.sync_copy(embs2.at[c, pl.ds(r0, 64)], pv)
        else:
          pltpu.sync_copy(accb.at[c, pl.ds(r0, 64)], pv)

        @pl.loop(0, 4)
        def _grp(gi):
          o = pl.multiple_of(gi * 16, 16)
          dvec = disc[pl.ds(o, 16)]
          for l in range(16):
            n = o + l
            d16 = _bcast_row(dvec, l)
            for j in range(H // 16):
              sl = pl.ds(j * 16, 16)
              a = av[n, sl]
              x = a * d16
              if final:
                pv[n, sl] = (pv[n, sl] + x) * 0.25
              else:
                pv[n, sl] = pv[n, sl] + x
                av[n, sl] = x * d16

        if final:
          pltpu.sync_copy(pv, out_h.at[c, pl.ds(r0, 64)])
        else:
          pltpu.sync_copy(pv, accb.at[c, pl.ds(r0, 64)])
          pltpu.sync_copy(av, tc.at[pl.ds(r0, 64)])

    plsc.subcore_barrier()

    if not final:
      zero_acc()
      plsc.subcore_barrier()


_fused = pl.kernel(
    _fused_body,
    out_type=[
        jax.ShapeDtypeStruct((NC, NACC, H), jnp.float32),
        jax.ShapeDtypeStruct((NC, NACC, H), jnp.float32),
        jax.ShapeDtypeStruct((NC, NACC, H), jnp.float32),
    ],
    mesh=_mesh,
    compiler_params=_params,
    scratch_types=[
        pltpu.VMEM_SHARED((NDIS,), jnp.float32),
        pltpu.VMEM_SHARED((NACC, H), jnp.float32),
        pltpu.VMEM((KMAX, CE), jnp.int32),
        pltpu.VMEM((KMAX, CE), jnp.int32),
        pltpu.VMEM((CE, H), jnp.float32),
        pltpu.VMEM((CE, H), jnp.float32),
        pltpu.VMEM((CE, H), jnp.float32),
        pltpu.VMEM((CE, H), jnp.float32),
        pltpu.VMEM((64, H), jnp.float32),
        pltpu.VMEM((64, H), jnp.float32),
        pltpu.VMEM((SL,), jnp.float32),
        pltpu.VMEM((64,), jnp.float32),
        pltpu.VMEM((CE,), jnp.float32),
        pltpu.SemaphoreType.DMA,
        pltpu.SemaphoreType.DMA,
        pltpu.SemaphoreType.DMA,
        pltpu.SemaphoreType.DMA,
        pltpu.SemaphoreType.DMA,
        pltpu.SemaphoreType.DMA,
    ],
)


def kernel(embs, edge_index):
  row = edge_index[0]
  col = edge_index[1]
  npad = EP - E
  ar = jnp.arange(npad, dtype=jnp.int32)
  padrow = (ar * 97) % N
  padcol = N + (ar % (NACC - N))
  row2 = jnp.concatenate([row, padrow]).reshape(NCHUNKP, CE)
  col2 = jnp.concatenate([col, padcol]).reshape(NCHUNKP, CE)
  embs2 = jnp.stack([embs[:, :H], embs[:, H:]])
  embs2 = jnp.pad(embs2, ((0, 0), (0, NACC - N), (0, 0)))
  out_h, _, _ = _fused(embs2, row2, col2)
  return jnp.concatenate([out_h[0, :N], out_h[1, :N]], axis=1)


# speedup vs baseline: 19.4753x; 1.0649x over previous
import jax
import jax.numpy as jnp
from jax import lax
from jax.experimental import pallas as pl
from jax.experimental.pallas import tpu as pltpu
from jax.experimental.pallas import tpu_sc as plsc

N = 10000
E = 320000
D = 128
H = D // 2
NC = 2
NS = 16
NACC = 10112
NRCH = NACC // 128
NRC64 = NACC // 64
NDIS = 10240
SL = NDIS // NS
CE = 128
KMAX = 160
NCHUNKP = KMAX * NS
EP = NCHUNKP * CE

_mesh = plsc.VectorSubcoreMesh(core_axis_name="c", subcore_axis_name="s")
_params = pltpu.CompilerParams(use_tc_tiling_on_sc=False)


def _rsqrt16(d):
  i = lax.bitcast_convert_type(d, jnp.int32)
  i = jnp.int32(0x5F3759DF) - lax.shift_right_logical(i, 1)
  y = lax.bitcast_convert_type(i, jnp.float32)
  for _ in range(3):
    t = (d * 0.5) * y
    y = y * (1.5 - t * y)
  return y


def _bcast_row(dvec, n):
  return jnp.full((16,), dvec[n], jnp.float32)


def _fused_body(embs_p, row2, col2, out_p, tbuf, accb,
                deg_sh, acc_sh, rowv2, colv2, b0, b1, b2, b3, av, pv, zbuf,
                dbuf, disc, ones_v, g0, g1, g2, g3, ssem, dsem):
  c = lax.axis_index("c")
  s = lax.axis_index("s")
  tc = tbuf.at[c]
  bufs = (b0, b1, b2, b3)
  gsems = (g0, g1, g2, g3)

  def fire_gather(k, b):
    pltpu.async_copy(tc.at[rowv2.at[k]], bufs[b], gsems[b])

  def wait_gather(k, b):
    pltpu.make_async_copy(tc.at[rowv2.at[k]], bufs[b], gsems[b]).wait()

  def fire_scatter(k, b):
    pltpu.async_copy(bufs[b], acc_sh.at[colv2.at[k]], ssem, add=True)

  def drain_scatter(k, b):
    pltpu.make_async_copy(bufs[b], acc_sh.at[colv2.at[k]], ssem).wait()

  def zero_acc():
    for r in range(CE):
      for j in range(H // 16):
        b0[r, pl.ds(j * 16, 16)] = jnp.zeros((16,), jnp.float32)
    for q in range(5):
      cq = s + NS * q

      @pl.when(cq < NRCH)
      def _(cq=cq):
        pltpu.sync_copy(b0, acc_sh.at[pl.ds(cq * 128, CE)])

  pltpu.sync_copy(row2.at[pl.ds(s * KMAX, KMAX)], rowv2)
  pltpu.sync_copy(col2.at[pl.ds(s * KMAX, KMAX)], colv2)
  for j in range(SL // 16):
    dbuf[pl.ds(j * 16, 16)] = jnp.zeros((16,), jnp.float32)
  for j in range(CE // 16):
    ones_v[pl.ds(j * 16, 16)] = jnp.ones((16,), jnp.float32)
  pltpu.sync_copy(dbuf, deg_sh.at[pl.ds(s * SL, SL)])
  for r in range(64):
    for j in range(H // 16):
      zbuf[r, pl.ds(j * 16, 16)] = jnp.zeros((16,), jnp.float32)
  zero_acc()
  plsc.subcore_barrier()

  @pl.loop(0, KMAX, step=8)
  def _deg(k):
    for b in range(8):
      pltpu.async_copy(ones_v, deg_sh.at[colv2.at[k + b]], dsem, add=True)
    for b in range(8):
      pltpu.make_async_copy(ones_v, deg_sh.at[colv2.at[k]], dsem).wait()

  plsc.subcore_barrier()

  pltpu.sync_copy(deg_sh.at[pl.ds(s * SL, SL)], dbuf)
  for j in range(SL // 16):
    d = dbuf[pl.ds(j * 16, 16)]
    y = jnp.where(d > 0, _rsqrt16(d), 0.0)
    dbuf[pl.ds(j * 16, 16)] = y
  pltpu.sync_copy(dbuf, deg_sh.at[pl.ds(s * SL, SL)])
  plsc.subcore_barrier()

  @pl.loop(0, 10)
  def _t0(q):
    cq = s + NS * q

    @pl.when(cq < NRC64)
    def _():
      r0 = cq * 64
      pltpu.sync_copy(embs_p.at[pl.ds(r0, 64), pl.ds(c * H, H)], av)
      pltpu.sync_copy(deg_sh.at[pl.ds(r0, 64)], disc)

      @pl.loop(0, 4)
      def _grp(gi):
        o = pl.multiple_of(gi * 16, 16)
        dvec = disc[pl.ds(o, 16)]
        for l in range(16):
          n = o + l
          d16 = _bcast_row(dvec, l)
          for j in range(H // 16):
            sl = pl.ds(j * 16, 16)
            av[n, sl] = av[n, sl] * d16

      pltpu.sync_copy(av, tc.at[pl.ds(r0, 64)])

  plsc.subcore_barrier()

  for li in range(3):
    final = li == 2

    for b in range(2):
      fire_gather(b, b)

    @pl.loop(0, KMAX, step=4)
    def _edges(g):
      wait_gather(g, 0)
      wait_gather(g + 1, 1)
      fire_scatter(g, 0)
      fire_scatter(g + 1, 1)
      fire_gather(g + 2, 2)
      fire_gather(g + 3, 3)
      drain_scatter(g, 0)
      drain_scatter(g + 1, 1)
      wait_gather(g + 2, 2)
      wait_gather(g + 3, 3)
      fire_scatter(g + 2, 2)
      fire_scatter(g + 3, 3)

      @pl.when(g + 4 < KMAX)
      def _():
        fire_gather(g + 4, 0)
        fire_gather(g + 5, 1)

      drain_scatter(g + 2, 2)
      drain_scatter(g + 3, 3)

    plsc.subcore_barrier()

    @pl.loop(0, 10)
    def _rows(q):
      cq = s + NS * q

      @pl.when(cq < NRC64)
      def _():
        r0 = cq * 64
        pltpu.sync_copy(acc_sh.at[pl.ds(r0, 64)], av)
        if not final:
          pltpu.sync_copy(zbuf, acc_sh.at[pl.ds(r0, 64)])
        pltpu.sync_copy(deg_sh.at[pl.ds(r0, 64)], disc)
        if li == 0:
          pltpu.sync_copy(embs_p.at[pl.ds(r0, 64), pl.ds(c * H, H)], pv)
        else:
          pltpu.sync_copy(accb.at[c, pl.ds(r0, 64)], pv)

        @pl.loop(0, 4)
        def _grp(gi):
          o = pl.multiple_of(gi * 16, 16)
          dvec = disc[pl.ds(o, 16)]
          for l in range(16):
            n = o + l
            d16 = _bcast_row(dvec, l)
            for j in range(H // 16):
              sl = pl.ds(j * 16, 16)
              a = av[n, sl]
              x = a * d16
              if final:
                pv[n, sl] = (pv[n, sl] + x) * 0.25
              else:
                pv[n, sl] = pv[n, sl] + x
                av[n, sl] = x * d16

        if final:
          pltpu.sync_copy(pv, out_p.at[pl.ds(r0, 64), pl.ds(c * H, H)])
        else:
          pltpu.sync_copy(pv, accb.at[c, pl.ds(r0, 64)])
          pltpu.sync_copy(av, tc.at[pl.ds(r0, 64)])

    plsc.subcore_barrier()


_fused = pl.kernel(
    _fused_body,
    out_type=[
        jax.ShapeDtypeStruct((NACC, D), jnp.float32),
        jax.ShapeDtypeStruct((NC, NACC, H), jnp.float32),
        jax.ShapeDtypeStruct((NC, NACC, H), jnp.float32),
    ],
    mesh=_mesh,
    compiler_params=_params,
    scratch_types=[
        pltpu.VMEM_SHARED((NDIS,), jnp.float32),
        pltpu.VMEM_SHARED((NACC, H), jnp.float32),
        pltpu.VMEM((KMAX, CE), jnp.int32),
        pltpu.VMEM((KMAX, CE), jnp.int32),
        pltpu.VMEM((CE, H), jnp.float32),
        pltpu.VMEM((CE, H), jnp.float32),
        pltpu.VMEM((CE, H), jnp.float32),
        pltpu.VMEM((CE, H), jnp.float32),
        pltpu.VMEM((64, H), jnp.float32),
        pltpu.VMEM((64, H), jnp.float32),
        pltpu.VMEM((64, H), jnp.float32),
        pltpu.VMEM((SL,), jnp.float32),
        pltpu.VMEM((64,), jnp.float32),
        pltpu.VMEM((CE,), jnp.float32),
        pltpu.SemaphoreType.DMA,
        pltpu.SemaphoreType.DMA,
        pltpu.SemaphoreType.DMA,
        pltpu.SemaphoreType.DMA,
        pltpu.SemaphoreType.DMA,
        pltpu.SemaphoreType.DMA,
    ],
)


def kernel(embs, edge_index):
  row = edge_index[0]
  col = edge_index[1]
  npad = EP - E
  ar = jnp.arange(npad, dtype=jnp.int32)
  padrow = (ar * 97) % N
  padcol = N + (ar % (NACC - N))
  row2 = jnp.concatenate([row, padrow]).reshape(NCHUNKP, CE)
  col2 = jnp.concatenate([col, padcol]).reshape(NCHUNKP, CE)
  embs_p = jnp.pad(embs, ((0, NACC - N), (0, 0)))
  out_p, _, _ = _fused(embs_p, row2, col2)
  return out_p[:N]


# speedup vs baseline: 20.6370x; 1.0596x over previous
import jax
import jax.numpy as jnp
from jax import lax
from jax.experimental import pallas as pl
from jax.experimental.pallas import tpu as pltpu
from jax.experimental.pallas import tpu_sc as plsc

N = 10000
E = 320000
D = 128
H = D // 2
NC = 2
NS = 16
NACC = 10112
NRCH = NACC // 128
NRC64 = NACC // 64
NDIS = 10240
SL = NDIS // NS
CE = 128
KMAX = 160
NCHUNKP = KMAX * NS
EP = NCHUNKP * CE

_mesh = plsc.VectorSubcoreMesh(core_axis_name="c", subcore_axis_name="s")
_params = pltpu.CompilerParams(use_tc_tiling_on_sc=False)


def _rsqrt16(d):
  i = lax.bitcast_convert_type(d, jnp.int32)
  i = jnp.int32(0x5F3759DF) - lax.shift_right_logical(i, 1)
  y = lax.bitcast_convert_type(i, jnp.float32)
  for _ in range(3):
    t = (d * 0.5) * y
    y = y * (1.5 - t * y)
  return y


def _bcast_row(dvec, n):
  return jnp.full((16,), dvec[n], jnp.float32)


def _fused_body(embs_p, row2, col2, out_p, tbuf, accb,
                deg_sh, acc_sh, rowv2, colv2, b0, b1, b2, b3, av, pv, zbuf,
                dbuf, disc, ones_v, g0, g1, g2, g3, ssem, dsem):
  c = lax.axis_index("c")
  s = lax.axis_index("s")
  tc = tbuf.at[c]
  bufs = (b0, b1, b2, b3)
  gsems = (g0, g1, g2, g3)

  def fire_gather(k, b):
    pltpu.async_copy(tc.at[rowv2.at[k]], bufs[b], gsems[b])

  def wait_gather(k, b):
    pltpu.make_async_copy(tc.at[rowv2.at[k]], bufs[b], gsems[b]).wait()

  def fire_scatter(k, b):
    pltpu.async_copy(bufs[b], acc_sh.at[colv2.at[k]], ssem, add=True)

  def drain_scatter(k, b):
    pltpu.make_async_copy(bufs[b], acc_sh.at[colv2.at[k]], ssem).wait()

  def zero_acc():
    for r in range(CE):
      for j in range(H // 16):
        b0[r, pl.ds(j * 16, 16)] = jnp.zeros((16,), jnp.float32)
    for q in range(5):
      cq = s + NS * q

      @pl.when(cq < NRCH)
      def _(cq=cq):
        pltpu.sync_copy(b0, acc_sh.at[pl.ds(cq * 128, CE)])

  pltpu.sync_copy(row2.at[pl.ds(s * KMAX, KMAX)], rowv2)
  pltpu.sync_copy(col2.at[pl.ds(s * KMAX, KMAX)], colv2)
  for j in range(SL // 16):
    dbuf[pl.ds(j * 16, 16)] = jnp.zeros((16,), jnp.float32)
  for j in range(CE // 16):
    ones_v[pl.ds(j * 16, 16)] = jnp.ones((16,), jnp.float32)
  pltpu.sync_copy(dbuf, deg_sh.at[pl.ds(s * SL, SL)])
  for r in range(64):
    for j in range(H // 16):
      zbuf[r, pl.ds(j * 16, 16)] = jnp.zeros((16,), jnp.float32)
  zero_acc()
  plsc.subcore_barrier()

  @pl.loop(0, KMAX, step=8)
  def _deg(k):
    for b in range(8):
      pltpu.async_copy(ones_v, deg_sh.at[colv2.at[k + b]], dsem, add=True)
    for b in range(8):
      pltpu.make_async_copy(ones_v, deg_sh.at[colv2.at[k]], dsem).wait()

  plsc.subcore_barrier()

  pltpu.sync_copy(deg_sh.at[pl.ds(s * SL, SL)], dbuf)
  for j in range(SL // 16):
    d = dbuf[pl.ds(j * 16, 16)]
    y = jnp.where(d > 0, _rsqrt16(d), 0.0)
    dbuf[pl.ds(j * 16, 16)] = y
  pltpu.sync_copy(dbuf, deg_sh.at[pl.ds(s * SL, SL)])
  plsc.subcore_barrier()

  @pl.loop(0, 10)
  def _t0(q):
    cq = s + NS * q

    @pl.when(cq < NRC64)
    def _():
      r0 = cq * 64
      pltpu.sync_copy(embs_p.at[pl.ds(r0, 64), pl.ds(c * H, H)], av)
      pltpu.sync_copy(deg_sh.at[pl.ds(r0, 64)], disc)

      @pl.loop(0, 4)
      def _grp(gi):
        o = pl.multiple_of(gi * 16, 16)
        dvec = disc[pl.ds(o, 16)]
        for l in range(16):
          n = o + l
          d16 = _bcast_row(dvec, l)
          for j in range(H // 16):
            sl = pl.ds(j * 16, 16)
            av[n, sl] = av[n, sl] * d16

      pltpu.sync_copy(av, tc.at[pl.ds(r0, 64)])

  plsc.subcore_barrier()

  for li in range(3):
    final = li == 2

    for b in range(2):
      fire_gather(b, b)

    @pl.loop(0, KMAX, step=4)
    def _edges(g):
      fire_gather(g + 2, 2)
      fire_gather(g + 3, 3)
      wait_gather(g, 0)
      wait_gather(g + 1, 1)
      fire_scatter(g, 0)
      fire_scatter(g + 1, 1)
      wait_gather(g + 2, 2)
      wait_gather(g + 3, 3)
      fire_scatter(g + 2, 2)
      fire_scatter(g + 3, 3)
      drain_scatter(g, 0)
      drain_scatter(g + 1, 1)

      @pl.when(g + 4 < KMAX)
      def _():
        fire_gather(g + 4, 0)
        fire_gather(g + 5, 1)

      drain_scatter(g + 2, 2)
      drain_scatter(g + 3, 3)

    plsc.subcore_barrier()

    @pl.loop(0, 10)
    def _rows(q):
      cq = s + NS * q

      @pl.when(cq < NRC64)
      def _():
        r0 = cq * 64
        pltpu.sync_copy(acc_sh.at[pl.ds(r0, 64)], av)
        if not final:
          pltpu.sync_copy(zbuf, acc_sh.at[pl.ds(r0, 64)])
        pltpu.sync_copy(deg_sh.at[pl.ds(r0, 64)], disc)
        if li == 0:
          pltpu.sync_copy(embs_p.at[pl.ds(r0, 64), pl.ds(c * H, H)], pv)
        else:
          pltpu.sync_copy(accb.at[c, pl.ds(r0, 64)], pv)

        @pl.loop(0, 4)
        def _grp(gi):
          o = pl.multiple_of(gi * 16, 16)
          dvec = disc[pl.ds(o, 16)]
          for l in range(16):
            n = o + l
            d16 = _bcast_row(dvec, l)
            for j in range(H // 16):
              sl = pl.ds(j * 16, 16)
              a = av[n, sl]
              x = a * d16
              if final:
                pv[n, sl] = (pv[n, sl] + x) * 0.25
              else:
                pv[n, sl] = pv[n, sl] + x
                av[n, sl] = x * d16

        if final:
          pltpu.sync_copy(pv, out_p.at[pl.ds(r0, 64), pl.ds(c * H, H)])
        else:
          pltpu.sync_copy(pv, accb.at[c, pl.ds(r0, 64)])
          pltpu.sync_copy(av, tc.at[pl.ds(r0, 64)])

    plsc.subcore_barrier()


_fused = pl.kernel(
    _fused_body,
    out_type=[
        jax.ShapeDtypeStruct((NACC, D), jnp.float32),
        jax.ShapeDtypeStruct((NC, NACC, H), jnp.float32),
        jax.ShapeDtypeStruct((NC, NACC, H), jnp.float32),
    ],
    mesh=_mesh,
    compiler_params=_params,
    scratch_types=[
        pltpu.VMEM_SHARED((NDIS,), jnp.float32),
        pltpu.VMEM_SHARED((NACC, H), jnp.float32),
        pltpu.VMEM((KMAX, CE), jnp.int32),
        pltpu.VMEM((KMAX, CE), jnp.int32),
        pltpu.VMEM((CE, H), jnp.float32),
        pltpu.VMEM((CE, H), jnp.float32),
        pltpu.VMEM((CE, H), jnp.float32),
        pltpu.VMEM((CE, H), jnp.float32),
        pltpu.VMEM((64, H), jnp.float32),
        pltpu.VMEM((64, H), jnp.float32),
        pltpu.VMEM((64, H), jnp.float32),
        pltpu.VMEM((SL,), jnp.float32),
        pltpu.VMEM((64,), jnp.float32),
        pltpu.VMEM((CE,), jnp.float32),
        pltpu.SemaphoreType.DMA,
        pltpu.SemaphoreType.DMA,
        pltpu.SemaphoreType.DMA,
        pltpu.SemaphoreType.DMA,
        pltpu.SemaphoreType.DMA,
        pltpu.SemaphoreType.DMA,
    ],
)


def kernel(embs, edge_index):
  row = edge_index[0]
  col = edge_index[1]
  npad = EP - E
  ar = jnp.arange(npad, dtype=jnp.int32)
  padrow = (ar * 97) % N
  padcol = N + (ar % (NACC - N))
  row2 = jnp.concatenate([row, padrow]).reshape(NCHUNKP, CE)
  col2 = jnp.concatenate([col, padcol]).reshape(NCHUNKP, CE)
  embs_p = jnp.pad(embs, ((0, NACC - N), (0, 0)))
  out_p, _, _ = _fused(embs_p, row2, col2)
  return out_p[:N]


# speedup vs baseline: 20.7125x; 1.0037x over previous
import jax
import jax.numpy as jnp
from jax import lax
from jax.experimental import pallas as pl
from jax.experimental.pallas import tpu as pltpu
from jax.experimental.pallas import tpu_sc as plsc

N = 10000
E = 320000
D = 128
H = D // 2
NC = 2
NS = 16
NACC = 10112
NRCH = NACC // 128
NRC64 = NACC // 64
NDIS = 10240
SL = NDIS // NS
CE = 128
KMAX = 160
NCHUNKP = KMAX * NS
EP = NCHUNKP * CE

_mesh = plsc.VectorSubcoreMesh(core_axis_name="c", subcore_axis_name="s")
_params = pltpu.CompilerParams(use_tc_tiling_on_sc=False)


def _rsqrt16(d):
  i = lax.bitcast_convert_type(d, jnp.int32)
  i = jnp.int32(0x5F3759DF) - lax.shift_right_logical(i, 1)
  y = lax.bitcast_convert_type(i, jnp.float32)
  for _ in range(3):
    t = (d * 0.5) * y
    y = y * (1.5 - t * y)
  return y


def _bcast_row(dvec, n):
  return jnp.full((16,), dvec[n], jnp.float32)


def _fused_body(embs_p, row2, col2, out_p, tbuf, accb,
                deg_sh, acc_sh, rowv2, colv2, b0, b1, b2, b3, av, pv, zbuf,
                dbuf, disc, ones_v, g0, g1, g2, g3, ssem, dsem):
  c = lax.axis_index("c")
  s = lax.axis_index("s")
  tc = tbuf.at[c]
  bufs = (b0, b1, b2, b3)
  gsems = (g0, g1, g2, g3)

  def fire_gather(k, b):
    pltpu.async_copy(tc.at[rowv2.at[k]], bufs[b], gsems[b])

  def wait_gather(k, b):
    pltpu.make_async_copy(tc.at[rowv2.at[k]], bufs[b], gsems[b]).wait()

  def fire_scatter(k, b):
    pltpu.async_copy(bufs[b], acc_sh.at[colv2.at[k]], ssem, add=True)

  def drain_scatter(k, b):
    pltpu.make_async_copy(bufs[b], acc_sh.at[colv2.at[k]], ssem).wait()

  def zero_acc():
    for r in range(CE):
      for j in range(H // 16):
        b0[r, pl.ds(j * 16, 16)] = jnp.zeros((16,), jnp.float32)
    for q in range(5):
      cq = s + NS * q

      @pl.when(cq < NRCH)
      def _(cq=cq):
        pltpu.sync_copy(b0, acc_sh.at[pl.ds(cq * 128, CE)])

  pltpu.sync_copy(row2.at[pl.ds(s * KMAX, KMAX)], rowv2)
  pltpu.sync_copy(col2.at[pl.ds(s * KMAX, KMAX)], colv2)
  for j in range(SL // 16):
    dbuf[pl.ds(j * 16, 16)] = jnp.zeros((16,), jnp.float32)
  for j in range(CE // 16):
    ones_v[pl.ds(j * 16, 16)] = jnp.ones((16,), jnp.float32)
  pltpu.sync_copy(dbuf, deg_sh.at[pl.ds(s * SL, SL)])
  for r in range(64):
    for j in range(H // 16):
      zbuf[r, pl.ds(j * 16, 16)] = jnp.zeros((16,), jnp.float32)
  zero_acc()
  plsc.subcore_barrier()

  @pl.loop(0, KMAX, step=8)
  def _deg(k):
    for b in range(8):
      pltpu.async_copy(ones_v, deg_sh.at[colv2.at[k + b]], dsem, add=True)
    for b in range(8):
      pltpu.make_async_copy(ones_v, deg_sh.at[colv2.at[k]], dsem).wait()

  plsc.subcore_barrier()

  pltpu.sync_copy(deg_sh.at[pl.ds(s * SL, SL)], dbuf)
  for j in range(SL // 16):
    d = dbuf[pl.ds(j * 16, 16)]
    y = jnp.where(d > 0, _rsqrt16(d), 0.0)
    dbuf[pl.ds(j * 16, 16)] = y
  pltpu.sync_copy(dbuf, deg_sh.at[pl.ds(s * SL, SL)])
  plsc.subcore_barrier()

  @pl.loop(0, 10)
  def _t0(q):
    cq = s + NS * q

    @pl.when(cq < NRC64)
    def _():
      r0 = cq * 64
      pltpu.sync_copy(embs_p.at[pl.ds(r0, 64), pl.ds(c * H, H)], av)
      pltpu.sync_copy(deg_sh.at[pl.ds(r0, 64)], disc)

      @pl.loop(0, 4)
      def _grp(gi):
        o = pl.multiple_of(gi * 16, 16)
        dvec = disc[pl.ds(o, 16)]
        for l in range(16):
          n = o + l
          d16 = _bcast_row(dvec, l)
          for j in range(H // 16):
            sl = pl.ds(j * 16, 16)
            av[n, sl] = av[n, sl] * d16

      pltpu.sync_copy(av, tc.at[pl.ds(r0, 64)])

  plsc.subcore_barrier()

  for li in range(3):
    final = li == 2

    for b in range(2):
      fire_gather(b, b)

    @pl.loop(0, KMAX, step=4)
    def _edges(g):
      fire_gather(g + 2, 2)
      fire_gather(g + 3, 3)
      wait_gather(g, 0)
      fire_scatter(g, 0)
      wait_gather(g + 1, 1)
      fire_scatter(g + 1, 1)
      wait_gather(g + 2, 2)
      fire_scatter(g + 2, 2)
      wait_gather(g + 3, 3)
      fire_scatter(g + 3, 3)
      drain_scatter(g, 0)
      drain_scatter(g + 1, 1)

      @pl.when(g + 4 < KMAX)
      def _():
        fire_gather(g + 4, 0)
        fire_gather(g + 5, 1)

      drain_scatter(g + 2, 2)
      drain_scatter(g + 3, 3)

    plsc.subcore_barrier()

    @pl.loop(0, 10)
    def _rows(q):
      cq = s + NS * q

      @pl.when(cq < NRC64)
      def _():
        r0 = cq * 64
        pltpu.sync_copy(acc_sh.at[pl.ds(r0, 64)], av)
        if not final:
          pltpu.sync_copy(zbuf, acc_sh.at[pl.ds(r0, 64)])
        pltpu.sync_copy(deg_sh.at[pl.ds(r0, 64)], disc)
        if li == 0:
          pltpu.sync_copy(embs_p.at[pl.ds(r0, 64), pl.ds(c * H, H)], pv)
        else:
          pltpu.sync_copy(accb.at[c, pl.ds(r0, 64)], pv)

        @pl.loop(0, 4)
        def _grp(gi):
          o = pl.multiple_of(gi * 16, 16)
          dvec = disc[pl.ds(o, 16)]
          for l in range(16):
            n = o + l
            d16 = _bcast_row(dvec, l)
            for j in range(H // 16):
              sl = pl.ds(j * 16, 16)
              a = av[n, sl]
              x = a * d16
              if final:
                pv[n, sl] = (pv[n, sl] + x) * 0.25
              else:
                pv[n, sl] = pv[n, sl] + x
                av[n, sl] = x * d16

        if final:
          pltpu.sync_copy(pv, out_p.at[pl.ds(r0, 64), pl.ds(c * H, H)])
        else:
          pltpu.sync_copy(pv, accb.at[c, pl.ds(r0, 64)])
          pltpu.sync_copy(av, tc.at[pl.ds(r0, 64)])

    plsc.subcore_barrier()


_fused = pl.kernel(
    _fused_body,
    out_type=[
        jax.ShapeDtypeStruct((NACC, D), jnp.float32),
        jax.ShapeDtypeStruct((NC, NACC, H), jnp.float32),
        jax.ShapeDtypeStruct((NC, NACC, H), jnp.float32),
    ],
    mesh=_mesh,
    compiler_params=_params,
    scratch_types=[
        pltpu.VMEM_SHARED((NDIS,), jnp.float32),
        pltpu.VMEM_SHARED((NACC, H), jnp.float32),
        pltpu.VMEM((KMAX, CE), jnp.int32),
        pltpu.VMEM((KMAX, CE), jnp.int32),
        pltpu.VMEM((CE, H), jnp.float32),
        pltpu.VMEM((CE, H), jnp.float32),
        pltpu.VMEM((CE, H), jnp.float32),
        pltpu.VMEM((CE, H), jnp.float32),
        pltpu.VMEM((64, H), jnp.float32),
        pltpu.VMEM((64, H), jnp.float32),
        pltpu.VMEM((64, H), jnp.float32),
        pltpu.VMEM((SL,), jnp.float32),
        pltpu.VMEM((64,), jnp.float32),
        pltpu.VMEM((CE,), jnp.float32),
        pltpu.SemaphoreType.DMA,
        pltpu.SemaphoreType.DMA,
        pltpu.SemaphoreType.DMA,
        pltpu.SemaphoreType.DMA,
        pltpu.SemaphoreType.DMA,
        pltpu.SemaphoreType.DMA,
    ],
)


def kernel(embs, edge_index):
  row = edge_index[0]
  col = edge_index[1]
  npad = EP - E
  ar = jnp.arange(npad, dtype=jnp.int32)
  padrow = (ar * 97) % N
  padcol = N + (ar % (NACC - N))
  row2 = jnp.concatenate([row, padrow]).reshape(NCHUNKP, CE)
  col2 = jnp.concatenate([col, padcol]).reshape(NCHUNKP, CE)
  embs_p = jnp.pad(embs, ((0, NACC - N), (0, 0)))
  out_p, _, _ = _fused(embs_p, row2, col2)
  return out_p[:N]
